# baseline (speedup 1.0000x reference)
import functools

import jax
import jax.numpy as jnp
from jax import lax
from jax.experimental import pallas as pl
from jax.experimental.pallas import tpu as pltpu
from jax.experimental.pallas import tpu_sc as plsc

NUM_SEQ = 16384
SEQ = 50
NUM_ROWS = NUM_SEQ * SEQ
DIM = 64
NC = 2
NS = 16
NW = NC * NS
PER_W = NUM_ROWS // NW
SEQ_PER_W = NUM_SEQ // NW
CHUNK = 2 * SEQ
N_CHUNKS = PER_W // CHUNK
NBUF = 8
N_GROUPS = N_CHUNKS // NBUF

_mesh = plsc.VectorSubcoreMesh(core_axis_name="c", subcore_axis_name="s")


@functools.partial(
    pl.kernel,
    mesh=_mesh,
    out_type=jax.ShapeDtypeStruct((NUM_SEQ, SEQ, DIM), jnp.float32),
    scratch_types=[
        pltpu.VMEM((N_CHUNKS, CHUNK), jnp.int32),
        pltpu.VMEM((NBUF, CHUNK, DIM), jnp.float32),
        pltpu.SemaphoreType.DMA((NBUF,)),
        pltpu.SemaphoreType.DMA((NBUF,)),
    ],
    compiler_params=pltpu.CompilerParams(use_tc_tiling_on_sc=False),
)
def _gather_kernel(idx_hbm, table_hbm, out_hbm, idx_v, rows_v, sem_g, sem_w):
    wid = lax.axis_index("s") * NC + lax.axis_index("c")
    chunk0 = pl.multiple_of(wid * N_CHUNKS, 8)
    seq0 = pl.multiple_of(wid * SEQ_PER_W, 8)
    pltpu.sync_copy(idx_hbm.at[pl.ds(chunk0, N_CHUNKS)], idx_v)

    def start_gather(b, j):
        pltpu.make_async_copy(
            table_hbm.at[idx_v.at[j]], rows_v.at[b], sem_g.at[b]
        ).start()

    def wait_gather(b):
        pltpu.make_async_copy(
            table_hbm.at[idx_v.at[0]], rows_v.at[b], sem_g.at[b]
        ).wait()

    def start_write(b, j):
        s = seq0 + 2 * j
        pltpu.make_async_copy(
            rows_v.at[b, pl.ds(0, SEQ)], out_hbm.at[s], sem_w.at[b]
        ).start()
        pltpu.make_async_copy(
            rows_v.at[b, pl.ds(SEQ, SEQ)], out_hbm.at[s + 1], sem_w.at[b]
        ).start()

    def wait_write(b, j):
        s = seq0 + 2 * j
        pltpu.make_async_copy(
            rows_v.at[b, pl.ds(0, SEQ)], out_hbm.at[s], sem_w.at[b]
        ).wait()
        pltpu.make_async_copy(
            rows_v.at[b, pl.ds(SEQ, SEQ)], out_hbm.at[s + 1], sem_w.at[b]
        ).wait()

    for b in range(NBUF):
        start_gather(b, b)

    def body(g, carry):
        for b in range(NBUF):
            j = g * NBUF + b
            wait_gather(b)
            start_write(b, j)
        for b in range(NBUF):
            j = g * NBUF + b
            wait_write(b, j)
            start_gather(b, j + NBUF)
        return carry

    lax.fori_loop(0, N_GROUPS - 1, body, 0)

    g_last = N_GROUPS - 1
    for b in range(NBUF):
        j = g_last * NBUF + b
        wait_gather(b)
        start_write(b, j)
    for b in range(NBUF):
        j = g_last * NBUF + b
        wait_write(b, j)


def kernel(input_ids, weight):
    idx = input_ids.reshape(NUM_ROWS // CHUNK, CHUNK).astype(jnp.int32)
    return _gather_kernel(idx, weight)
